# baseline (speedup 1.0000x reference)
import functools

import jax
import jax.numpy as jnp
from jax import lax
from jax.experimental import pallas as pl
from jax.experimental.pallas import tpu as pltpu
from jax.experimental.pallas import tpu_sc as plsc

_B, _S, _V = 32, 8192, 100000
_L = 16
_NC, _NS = 2, 16
_UNROLL = 4


def _hist_body(ids_hbm, last_hbm, out_hbm, ids_v, last_v, hist_v, sem_ids):
    c = lax.axis_index("c")
    s = lax.axis_index("s")
    wid = s * _NC + c

    ids_cp = pltpu.async_copy(ids_hbm.at[wid], ids_v, sem_ids)
    pltpu.sync_copy(last_hbm, last_v.at[pl.ds(0, _B)])
    last_b = last_v[pl.ds(wid, _L)][0]

    zeros = jnp.zeros((_L,), jnp.int32)

    def _zero(i, carry):
        hist_v[pl.ds(i * _L, _L)] = zeros
        return carry

    lax.fori_loop(0, _V // _L, _zero, 0, unroll=8)
    ids_cp.wait()

    iota = lax.iota(jnp.int32, _L)
    ones = jnp.ones((_L,), jnp.int32)
    n_steps = (last_b + _L * _UNROLL - 1) // (_L * _UNROLL)

    def _scat(i, carry):
        for j in range(_UNROLL):
            base = (i * _UNROLL + j) * _L
            ids16 = ids_v[pl.ds(base, _L)]
            m = (iota + base) < last_b
            plsc.addupdate_scatter(hist_v, [ids16], ones, mask=m)
        return carry

    lax.fori_loop(0, n_steps, _scat, 0)

    pltpu.sync_copy(hist_v, out_hbm.at[wid])


@functools.partial(jax.jit, static_argnames=())
def kernel(input_ids, last_token_index):
    last_flat = last_token_index.reshape(_B).astype(jnp.int32)
    mesh = plsc.VectorSubcoreMesh(
        core_axis_name="c", subcore_axis_name="s",
        num_cores=_NC, num_subcores=_NS,
    )
    run = pl.kernel(
        _hist_body,
        out_type=jax.ShapeDtypeStruct((_B, _V), jnp.int32),
        mesh=mesh,
        compiler_params=pltpu.CompilerParams(
            needs_layout_passes=False,
            skip_device_barrier=True,
            disable_bounds_checks=True,
            disable_semaphore_checks=True,
        ),
        scratch_types=[
            pltpu.VMEM((_S,), jnp.int32),
            pltpu.VMEM((_B + _L,), jnp.int32),
            pltpu.VMEM((_V,), jnp.int32),
            pltpu.SemaphoreType.DMA,
        ],
    )
    return run(input_ids.astype(jnp.int32), last_flat)


# speedup vs baseline: 1.0198x; 1.0198x over previous
import functools

import jax
import jax.numpy as jnp
from jax import lax
from jax.experimental import pallas as pl
from jax.experimental.pallas import tpu as pltpu
from jax.experimental.pallas import tpu_sc as plsc

_B, _S, _V = 32, 8192, 100000
_L = 16
_NC, _NS = 2, 16
_UNROLL = 4


def _hist_body(ids_hbm, last_hbm, out_hbm, ids_v, last_v, hist_v, sem_ids,
               sem_last):
    c = lax.axis_index("c")
    s = lax.axis_index("s")
    wid = s * _NC + c

    ids_cp = pltpu.async_copy(ids_hbm.at[wid], ids_v, sem_ids)
    last_cp = pltpu.async_copy(last_hbm, last_v.at[pl.ds(0, _B)], sem_last)

    zeros = jnp.zeros((_L,), jnp.int32)

    def _zero(i, carry):
        hist_v[pl.ds(i * _L, _L)] = zeros
        return carry

    lax.fori_loop(0, _V // _L, _zero, 0, unroll=8)
    last_cp.wait()
    ids_cp.wait()
    last_b = last_v[pl.ds(wid, _L)][0]

    iota = lax.iota(jnp.int32, _L)
    ones = jnp.ones((_L,), jnp.int32)
    n_steps = (last_b + _L * _UNROLL - 1) // (_L * _UNROLL)

    def _scat(i, carry):
        for j in range(_UNROLL):
            base = (i * _UNROLL + j) * _L
            ids16 = ids_v[pl.ds(base, _L)]
            m = (iota + base) < last_b
            plsc.addupdate_scatter(hist_v, [ids16], ones, mask=m)
        return carry

    lax.fori_loop(0, n_steps, _scat, 0)

    pltpu.sync_copy(hist_v, out_hbm.at[wid])


@functools.partial(jax.jit, static_argnames=())
def kernel(input_ids, last_token_index):
    last_flat = last_token_index.reshape(_B).astype(jnp.int32)
    mesh = plsc.VectorSubcoreMesh(
        core_axis_name="c", subcore_axis_name="s",
        num_cores=_NC, num_subcores=_NS,
    )
    run = pl.kernel(
        _hist_body,
        out_type=jax.ShapeDtypeStruct((_B, _V), jnp.int32),
        mesh=mesh,
        compiler_params=pltpu.CompilerParams(needs_layout_passes=False),
        scratch_types=[
            pltpu.VMEM((_S,), jnp.int32),
            pltpu.VMEM((_B + _L,), jnp.int32),
            pltpu.VMEM((_V,), jnp.int32),
            pltpu.SemaphoreType.DMA,
            pltpu.SemaphoreType.DMA,
        ],
    )
    return run(input_ids.astype(jnp.int32), last_flat)
